# Initial kernel scaffold; baseline (speedup 1.0000x reference)
import jax
import jax.numpy as jnp
from jax.experimental import pallas as pl


def kernel(node_features, edge_features, edges, W1_e, W2_e, W1_n, W2_n):
    raise NotImplementedError("write your pallas kernel here")


# speedup vs baseline: 5.9977x; 5.9977x over previous
import jax
import jax.numpy as jnp
from jax import lax
from jax.experimental import pallas as pl
from jax.experimental.pallas import tpu as pltpu
from jax.experimental.pallas import tpu_sc as plsc

N_NODES = 10000
N_EDGES = 320000
D_FEAT = 128
D_EDGE = 16
HID = 128
N_FILTERS = 128

NC = 2
NS = 16
NW = NC * NS

NCHUNK = 5
EC = N_EDGES // NCHUNK


def _sc_mesh():
    return plsc.VectorSubcoreMesh(
        core_axis_name="c", subcore_axis_name="s", num_cores=NC, num_subcores=NS
    )


GB = EC // NS
GSUB = 80
GK = 5
GC = GSUB * GK
GNCH = GB // GC


def _gather_body(koff, table_hbm, src_hbm, dst_hbm, a_hbm, b_hbm, idx_all,
                 rows, gsem0, gsem1, ssem0, ssem1):
    c = lax.axis_index("c")
    s = lax.axis_index("s")
    base = s * GB
    gsems = (gsem0, gsem1)
    ssems = (ssem0, ssem1)

    def run(idx_hbm, out_hbm):
        pltpu.sync_copy(idx_hbm.at[pl.ds(koff + base, GB)], idx_all)

        def g_desc(i, b, k):
            off = i * GC + k * GSUB
            return pltpu.make_async_copy(
                table_hbm.at[idx_all.at[pl.ds(off, GSUB)]],
                rows.at[b, pl.ds(k * GSUB, GSUB)],
                gsems[b],
            )

        def s_desc(i, b):
            return pltpu.make_async_copy(
                rows.at[b], out_hbm.at[pl.ds(base + i * GC, GC)], ssems[b]
            )

        for k in range(GK):
            g_desc(0, 0, k).start()

        def outer(j, carry):
            for b in range(2):
                i = 2 * j + b
                nb = 1 - b

                @pl.when(i >= 1)
                def _():
                    s_desc(i - 1, nb).wait()

                @pl.when(i + 1 < GNCH)
                def _():
                    for k in range(GK):
                        g_desc(i + 1, nb, k).start()

                for k in range(GK):
                    g_desc(i, b, k).wait()
                s_desc(i, b).start()
            return carry

        lax.fori_loop(0, GNCH // 2, outer, None)
        s_desc(GNCH - 1, (GNCH - 1) % 2).wait()

    @pl.when(c == 0)
    def _():
        run(src_hbm, a_hbm)

    @pl.when(c == 1)
    def _():
        run(dst_hbm, b_hbm)


def _sc_gather(node_features, src1d, dst1d, k):
    import functools
    f = pl.kernel(
        functools.partial(_gather_body, k * EC),
        out_type=(
            jax.ShapeDtypeStruct((EC, D_FEAT), jnp.float32),
            jax.ShapeDtypeStruct((EC, D_FEAT), jnp.float32),
        ),
        mesh=_sc_mesh(),
        compiler_params=pltpu.CompilerParams(use_tc_tiling_on_sc=False),
        scratch_types=[
            pltpu.VMEM((GB,), jnp.int32),
            pltpu.VMEM((2, GC, D_FEAT), jnp.float32),
            pltpu.SemaphoreType.DMA,
            pltpu.SemaphoreType.DMA,
            pltpu.SemaphoreType.DMA,
            pltpu.SemaphoreType.DMA,
        ],
        name=f"edge_gather_{k}",
    )
    return f(node_features, src1d, dst1d)


SB = EC // NW
SSUB = 80
SKK = SB // SSUB
SUNR = 5
NPS = 624
NPS_LAST = N_NODES - (NS - 1) * NPS


def _scatter_body(koff, msg_hbm, src_hbm, dst_hbm, out_hbm, table, msgv, srcv,
                  dstv, zbuf, scsem):
    c = lax.axis_index("c")
    s = lax.axis_index("s")
    w = c * NS + s
    ebase = w * SB

    pltpu.sync_copy(msg_hbm.at[pl.ds(ebase, SB)], msgv)
    pltpu.sync_copy(src_hbm.at[pl.ds(koff + ebase, SB)], srcv)
    pltpu.sync_copy(dst_hbm.at[pl.ds(koff + ebase, SB)], dstv)

    def zloop(i, carry):
        zbuf[i, :] = jnp.zeros((D_EDGE,), jnp.float32)
        return carry

    lax.fori_loop(0, NPS_LAST, zloop, None)

    @pl.when(s < NS - 1)
    def _():
        pltpu.sync_copy(zbuf.at[pl.ds(0, NPS)], table.at[pl.ds(s * NPS, NPS)])

    @pl.when(s == NS - 1)
    def _():
        pltpu.sync_copy(zbuf, table.at[pl.ds((NS - 1) * NPS, NPS_LAST)])

    plsc.subcore_barrier()

    def sub(jj, c2):
        for u in range(SUNR):
            j = jj * SUNR + u
            moff = j * SSUB
            pltpu.async_copy(
                msgv.at[pl.ds(moff, SSUB)],
                table.at[srcv.at[pl.ds(moff, SSUB)]], scsem, add=True,
            )
            pltpu.async_copy(
                msgv.at[pl.ds(moff, SSUB)],
                table.at[dstv.at[pl.ds(moff, SSUB)]], scsem, add=True,
            )
        return c2

    lax.fori_loop(0, SKK // SUNR, sub, None)

    def subw(jj, c2):
        for u in range(SUNR):
            j = jj * SUNR + u
            moff = j * SSUB
            pltpu.make_async_copy(
                msgv.at[pl.ds(moff, SSUB)],
                table.at[srcv.at[pl.ds(moff, SSUB)]], scsem,
            ).wait()
            pltpu.make_async_copy(
                msgv.at[pl.ds(moff, SSUB)],
                table.at[dstv.at[pl.ds(moff, SSUB)]], scsem,
            ).wait()
        return c2

    lax.fori_loop(0, SKK // SUNR, subw, None)
    plsc.subcore_barrier()

    @pl.when(s < NS - 1)
    def _():
        pltpu.sync_copy(
            table.at[pl.ds(s * NPS, NPS)], out_hbm.at[c, pl.ds(s * NPS, NPS)]
        )

    @pl.when(s == NS - 1)
    def _():
        pltpu.sync_copy(
            table.at[pl.ds((NS - 1) * NPS, NPS_LAST)],
            out_hbm.at[c, pl.ds((NS - 1) * NPS, NPS_LAST)],
        )


def _sc_scatter(messages_k, src1d, dst1d, k):
    import functools
    f = pl.kernel(
        functools.partial(_scatter_body, k * EC),
        out_type=jax.ShapeDtypeStruct((NC, N_NODES, D_EDGE), jnp.float32),
        mesh=_sc_mesh(),
        compiler_params=pltpu.CompilerParams(use_tc_tiling_on_sc=False),
        scratch_types=[
            pltpu.VMEM_SHARED((N_NODES, D_EDGE), jnp.float32),
            pltpu.VMEM((SB, D_EDGE), jnp.float32),
            pltpu.VMEM((SB,), jnp.int32),
            pltpu.VMEM((SB,), jnp.int32),
            pltpu.VMEM((NPS_LAST, D_EDGE), jnp.float32),
            pltpu.SemaphoreType.DMA,
        ],
        name=f"msg_scatter_{k}",
    )
    return f(messages_k, src1d, dst1d)


EB = 2560


def _edge_mlp_body(a_ref, b_ref, e_ref, w1_ref, w2_ref, out_ref):
    a = a_ref[...]
    b = b_ref[...]
    sym = (0.5 * (a + b)).astype(jnp.bfloat16)
    asym = (0.5 * jnp.abs(a - b)).astype(jnp.bfloat16)
    w1 = w1_ref[...]
    w1b = w1.astype(jnp.bfloat16)
    h = (
        jnp.dot(sym, w1b[0:D_FEAT], preferred_element_type=jnp.float32)
        + jnp.dot(asym, w1b[D_FEAT:2 * D_FEAT], preferred_element_type=jnp.float32)
        + jnp.dot(e_ref[...].astype(jnp.bfloat16),
                  w1b[2 * D_FEAT:2 * D_FEAT + D_EDGE],
                  preferred_element_type=jnp.float32)
        + w1[2 * D_FEAT + D_EDGE]
    )
    h = h * jax.nn.sigmoid(h)
    w2 = w2_ref[...]
    y = (
        jnp.dot(h.astype(jnp.bfloat16), w2.astype(jnp.bfloat16)[:HID],
                preferred_element_type=jnp.float32)
        + w2[HID]
    )
    out_ref[...] = y * jax.nn.sigmoid(y)


def _edge_mlp(a_rows, b_rows, ef_k, W1_e, W2_e):
    return pl.pallas_call(
        _edge_mlp_body,
        grid=(EC // EB,),
        in_specs=[
            pl.BlockSpec((EB, D_FEAT), lambda i: (i, 0)),
            pl.BlockSpec((EB, D_FEAT), lambda i: (i, 0)),
            pl.BlockSpec((EB, D_EDGE), lambda i: (i, 0)),
            pl.BlockSpec(W1_e.shape, lambda i: (0, 0)),
            pl.BlockSpec(W2_e.shape, lambda i: (0, 0)),
        ],
        out_specs=pl.BlockSpec((EB, D_EDGE), lambda i: (i, 0)),
        out_shape=jax.ShapeDtypeStruct((EC, D_EDGE), jnp.float32),
    )(a_rows, b_rows, ef_k, W1_e, W2_e)


NB = 2000


def _node_mlp_body(nf_ref, p_ref, w1_ref, w2_ref, out_ref):
    upd = jnp.sum(p_ref[...], axis=(0, 1))
    x = nf_ref[...]
    w1 = w1_ref[...]
    h = (
        jnp.dot(x, w1[0:D_FEAT], preferred_element_type=jnp.float32)
        + jnp.dot(upd, w1[D_FEAT:D_FEAT + D_EDGE],
                  preferred_element_type=jnp.float32)
        + w1[D_FEAT + D_EDGE]
    )
    h = h * jax.nn.sigmoid(h)
    w2 = w2_ref[...]
    y = jnp.dot(h, w2[:HID], preferred_element_type=jnp.float32) + w2[HID]
    out_ref[...] = y * jax.nn.sigmoid(y)


def _node_mlp(node_features, partials, W1_n, W2_n):
    return pl.pallas_call(
        _node_mlp_body,
        grid=(N_NODES // NB,),
        in_specs=[
            pl.BlockSpec((NB, D_FEAT), lambda i: (i, 0)),
            pl.BlockSpec((NCHUNK, NC, NB, D_EDGE), lambda i: (0, 0, i, 0)),
            pl.BlockSpec(W1_n.shape, lambda i: (0, 0)),
            pl.BlockSpec(W2_n.shape, lambda i: (0, 0)),
        ],
        out_specs=pl.BlockSpec((NB, N_FILTERS), lambda i: (i, 0)),
        out_shape=jax.ShapeDtypeStruct((N_NODES, N_FILTERS), jnp.float32),
    )(node_features, partials, W1_n, W2_n)


def kernel(node_features, edge_features, edges, W1_e, W2_e, W1_n, W2_n):
    src1d = edges[:, 0]
    dst1d = edges[:, 1]
    msgs = []
    parts = []
    for k in range(NCHUNK):
        a_rows, b_rows = _sc_gather(node_features, src1d, dst1d, k)
        ef_k = lax.slice(edge_features, (k * EC, 0), ((k + 1) * EC, D_EDGE))
        msg_k = _edge_mlp(a_rows, b_rows, ef_k, W1_e, W2_e)
        msgs.append(msg_k)
        parts.append(_sc_scatter(msg_k, src1d, dst1d, k))
    messages = jnp.concatenate(msgs, axis=0)
    partials = jnp.stack(parts, axis=0)
    updated = _node_mlp(node_features, partials, W1_n, W2_n)
    return (updated, messages)


# speedup vs baseline: 6.2502x; 1.0421x over previous
import jax
import jax.numpy as jnp
from jax import lax
from jax.experimental import pallas as pl
from jax.experimental.pallas import tpu as pltpu
from jax.experimental.pallas import tpu_sc as plsc

N_NODES = 10000
N_EDGES = 320000
D_FEAT = 128
D_EDGE = 16
HID = 128
N_FILTERS = 128

NC = 2
NS = 16
NW = NC * NS

NCHUNK = 5
EC = N_EDGES // NCHUNK


def _sc_mesh():
    return plsc.VectorSubcoreMesh(
        core_axis_name="c", subcore_axis_name="s", num_cores=NC, num_subcores=NS
    )


DPK = D_FEAT // 2
GB = EC // NS
GSUB = 80
GK = 5
GC = GSUB * GK
GNCH = GB // GC
EHALF = EC // 2


def _gather_body(koff, table_hbm, src_hbm, dst_hbm, a_hbm, b_hbm, idx_all,
                 rows, gsem0, gsem1, ssem0, ssem1):
    c = lax.axis_index("c")
    s = lax.axis_index("s")
    base = s * GB
    row0 = (s % (NS // 2)) * GB
    col0 = (s // (NS // 2)) * DPK
    gsems = (gsem0, gsem1)
    ssems = (ssem0, ssem1)

    def run(idx_hbm, out_hbm):
        pltpu.sync_copy(idx_hbm.at[pl.ds(koff + base, GB)], idx_all)

        def g_desc(i, b, k):
            off = i * GC + k * GSUB
            return pltpu.make_async_copy(
                table_hbm.at[idx_all.at[pl.ds(off, GSUB)]],
                rows.at[b, pl.ds(k * GSUB, GSUB)],
                gsems[b],
            )

        def s_desc(i, b):
            return pltpu.make_async_copy(
                rows.at[b],
                out_hbm.at[pl.ds(row0 + i * GC, GC), pl.ds(col0, DPK)],
                ssems[b],
            )

        for k in range(GK):
            g_desc(0, 0, k).start()

        def outer(j, carry):
            for b in range(2):
                i = 2 * j + b
                nb = 1 - b

                @pl.when(i >= 1)
                def _():
                    s_desc(i - 1, nb).wait()

                @pl.when(i + 1 < GNCH)
                def _():
                    for k in range(GK):
                        g_desc(i + 1, nb, k).start()

                for k in range(GK):
                    g_desc(i, b, k).wait()
                s_desc(i, b).start()
            return carry

        lax.fori_loop(0, GNCH // 2, outer, None)
        s_desc(GNCH - 1, (GNCH - 1) % 2).wait()

    @pl.when(c == 0)
    def _():
        run(src_hbm, a_hbm)

    @pl.when(c == 1)
    def _():
        run(dst_hbm, b_hbm)


def _sc_gather(nf_packed, src1d, dst1d, k):
    import functools
    f = pl.kernel(
        functools.partial(_gather_body, k * EC),
        out_type=(
            jax.ShapeDtypeStruct((EHALF, 2 * DPK), jnp.int32),
            jax.ShapeDtypeStruct((EHALF, 2 * DPK), jnp.int32),
        ),
        mesh=_sc_mesh(),
        compiler_params=pltpu.CompilerParams(use_tc_tiling_on_sc=False),
        scratch_types=[
            pltpu.VMEM((GB,), jnp.int32),
            pltpu.VMEM((2, GC, DPK), jnp.int32),
            pltpu.SemaphoreType.DMA,
            pltpu.SemaphoreType.DMA,
            pltpu.SemaphoreType.DMA,
            pltpu.SemaphoreType.DMA,
        ],
        name=f"edge_gather_{k}",
    )
    return f(nf_packed, src1d, dst1d)


SB = EC // NW
SSUB = 80
SKK = SB // SSUB
SUNR = 5
NPS = 624
NPS_LAST = N_NODES - (NS - 1) * NPS


def _scatter_body(koff, msg_hbm, src_hbm, dst_hbm, out_hbm, table, msgv, srcv,
                  dstv, zbuf, scsem):
    c = lax.axis_index("c")
    s = lax.axis_index("s")
    w = c * NS + s
    ebase = w * SB

    pltpu.sync_copy(msg_hbm.at[pl.ds(ebase, SB)], msgv)
    pltpu.sync_copy(src_hbm.at[pl.ds(koff + ebase, SB)], srcv)
    pltpu.sync_copy(dst_hbm.at[pl.ds(koff + ebase, SB)], dstv)

    def zloop(i, carry):
        zbuf[i, :] = jnp.zeros((D_EDGE,), jnp.float32)
        return carry

    lax.fori_loop(0, NPS_LAST, zloop, None)

    @pl.when(s < NS - 1)
    def _():
        pltpu.sync_copy(zbuf.at[pl.ds(0, NPS)], table.at[pl.ds(s * NPS, NPS)])

    @pl.when(s == NS - 1)
    def _():
        pltpu.sync_copy(zbuf, table.at[pl.ds((NS - 1) * NPS, NPS_LAST)])

    plsc.subcore_barrier()

    def sub(jj, c2):
        for u in range(SUNR):
            j = jj * SUNR + u
            moff = j * SSUB
            pltpu.async_copy(
                msgv.at[pl.ds(moff, SSUB)],
                table.at[srcv.at[pl.ds(moff, SSUB)]], scsem, add=True,
            )
            pltpu.async_copy(
                msgv.at[pl.ds(moff, SSUB)],
                table.at[dstv.at[pl.ds(moff, SSUB)]], scsem, add=True,
            )
        return c2

    lax.fori_loop(0, SKK // SUNR, sub, None)

    def subw(jj, c2):
        for u in range(SUNR):
            j = jj * SUNR + u
            moff = j * SSUB
            pltpu.make_async_copy(
                msgv.at[pl.ds(moff, SSUB)],
                table.at[srcv.at[pl.ds(moff, SSUB)]], scsem,
            ).wait()
            pltpu.make_async_copy(
                msgv.at[pl.ds(moff, SSUB)],
                table.at[dstv.at[pl.ds(moff, SSUB)]], scsem,
            ).wait()
        return c2

    lax.fori_loop(0, SKK // SUNR, subw, None)
    plsc.subcore_barrier()

    @pl.when(s < NS - 1)
    def _():
        pltpu.sync_copy(
            table.at[pl.ds(s * NPS, NPS)], out_hbm.at[c, pl.ds(s * NPS, NPS)]
        )

    @pl.when(s == NS - 1)
    def _():
        pltpu.sync_copy(
            table.at[pl.ds((NS - 1) * NPS, NPS_LAST)],
            out_hbm.at[c, pl.ds((NS - 1) * NPS, NPS_LAST)],
        )


def _sc_scatter(messages_k, src1d, dst1d, k):
    import functools
    f = pl.kernel(
        functools.partial(_scatter_body, k * EC),
        out_type=jax.ShapeDtypeStruct((NC, N_NODES, D_EDGE), jnp.float32),
        mesh=_sc_mesh(),
        compiler_params=pltpu.CompilerParams(use_tc_tiling_on_sc=False),
        scratch_types=[
            pltpu.VMEM_SHARED((N_NODES, D_EDGE), jnp.float32),
            pltpu.VMEM((SB, D_EDGE), jnp.float32),
            pltpu.VMEM((SB,), jnp.int32),
            pltpu.VMEM((SB,), jnp.int32),
            pltpu.VMEM((NPS_LAST, D_EDGE), jnp.float32),
            pltpu.SemaphoreType.DMA,
        ],
        name=f"msg_scatter_{k}",
    )
    return f(messages_k, src1d, dst1d)


EB = 1280
NBLK = EHALF // EB


def _unpack(x32):
    evens = lax.bitcast_convert_type(x32 << 16, jnp.float32)
    odds = lax.bitcast_convert_type(x32 & jnp.int32(-65536), jnp.float32)
    return jnp.concatenate([evens, odds], axis=1)


def _edge_mlp_half(a32, b32, ef, w1, w1b, w2):
    a = _unpack(a32)
    b = _unpack(b32)
    sym = (0.5 * (a + b)).astype(jnp.bfloat16)
    asym = (0.5 * jnp.abs(a - b)).astype(jnp.bfloat16)
    h = (
        jnp.dot(sym, w1b[0:D_FEAT], preferred_element_type=jnp.float32)
        + jnp.dot(asym, w1b[D_FEAT:2 * D_FEAT], preferred_element_type=jnp.float32)
        + jnp.dot(ef.astype(jnp.bfloat16),
                  w1b[2 * D_FEAT:2 * D_FEAT + D_EDGE],
                  preferred_element_type=jnp.float32)
        + w1[2 * D_FEAT + D_EDGE]
    )
    h = h * jax.nn.sigmoid(h)
    y = (
        jnp.dot(h.astype(jnp.bfloat16), w2.astype(jnp.bfloat16)[:HID],
                preferred_element_type=jnp.float32)
        + w2[HID]
    )
    return y * jax.nn.sigmoid(y)


def _edge_mlp_body(a_ref, b_ref, e_ref, w1_ref, w2_ref, out_ref):
    a32 = a_ref[...]
    b32 = b_ref[...]
    ef = e_ref[...]
    w1 = w1_ref[...]
    w1b = w1.astype(jnp.bfloat16)
    w2 = w2_ref[...]
    out_ref[0] = _edge_mlp_half(
        a32[:, :DPK], b32[:, :DPK], ef[0], w1, w1b, w2
    )
    out_ref[1] = _edge_mlp_half(
        a32[:, DPK:], b32[:, DPK:], ef[1], w1, w1b, w2
    )


def _edge_mlp(a_rows, b_rows, ef_k2, W1p, W2_e):
    out = pl.pallas_call(
        _edge_mlp_body,
        grid=(NBLK,),
        in_specs=[
            pl.BlockSpec((EB, 2 * DPK), lambda i: (i, 0)),
            pl.BlockSpec((EB, 2 * DPK), lambda i: (i, 0)),
            pl.BlockSpec((2, EB, D_EDGE), lambda i: (0, i, 0)),
            pl.BlockSpec(W1p.shape, lambda i: (0, 0)),
            pl.BlockSpec(W2_e.shape, lambda i: (0, 0)),
        ],
        out_specs=pl.BlockSpec((2, EB, D_EDGE), lambda i: (0, i, 0)),
        out_shape=jax.ShapeDtypeStruct((2, EHALF, D_EDGE), jnp.float32),
    )(a_rows, b_rows, ef_k2, W1p, W2_e)
    return out.reshape(EC, D_EDGE)


NB = 2000


def _node_mlp_body(nf_ref, p_ref, w1_ref, w2_ref, out_ref):
    upd = jnp.sum(p_ref[...], axis=(0, 1))
    x = nf_ref[...]
    w1 = w1_ref[...]
    h = (
        jnp.dot(x, w1[0:D_FEAT], preferred_element_type=jnp.float32)
        + jnp.dot(upd, w1[D_FEAT:D_FEAT + D_EDGE],
                  preferred_element_type=jnp.float32)
        + w1[D_FEAT + D_EDGE]
    )
    h = h * jax.nn.sigmoid(h)
    w2 = w2_ref[...]
    y = jnp.dot(h, w2[:HID], preferred_element_type=jnp.float32) + w2[HID]
    out_ref[...] = y * jax.nn.sigmoid(y)


def _node_mlp(node_features, partials, W1_n, W2_n):
    return pl.pallas_call(
        _node_mlp_body,
        grid=(N_NODES // NB,),
        in_specs=[
            pl.BlockSpec((NB, D_FEAT), lambda i: (i, 0)),
            pl.BlockSpec((NCHUNK, NC, NB, D_EDGE), lambda i: (0, 0, i, 0)),
            pl.BlockSpec(W1_n.shape, lambda i: (0, 0)),
            pl.BlockSpec(W2_n.shape, lambda i: (0, 0)),
        ],
        out_specs=pl.BlockSpec((NB, N_FILTERS), lambda i: (i, 0)),
        out_shape=jax.ShapeDtypeStruct((N_NODES, N_FILTERS), jnp.float32),
    )(node_features, partials, W1_n, W2_n)


def kernel(node_features, edge_features, edges, W1_e, W2_e, W1_n, W2_n):
    src1d = edges[:, 0]
    dst1d = edges[:, 1]
    nf_packed = lax.bitcast_convert_type(
        node_features.astype(jnp.bfloat16).reshape(N_NODES, DPK, 2), jnp.int32
    )
    perm = jnp.concatenate(
        [jnp.arange(0, D_FEAT, 2), jnp.arange(1, D_FEAT, 2)]
    )
    W1p = jnp.concatenate(
        [W1_e[0:D_FEAT][perm], W1_e[D_FEAT:2 * D_FEAT][perm],
         W1_e[2 * D_FEAT:]], axis=0
    )
    msgs = []
    parts = []
    for k in range(NCHUNK):
        a_rows, b_rows = _sc_gather(nf_packed, src1d, dst1d, k)
        ef_k = lax.slice(edge_features, (k * EC, 0), ((k + 1) * EC, D_EDGE))
        msg_k = _edge_mlp(a_rows, b_rows, ef_k.reshape(2, EHALF, D_EDGE),
                          W1p, W2_e)
        msgs.append(msg_k)
        parts.append(_sc_scatter(msg_k, src1d, dst1d, k))
    messages = jnp.concatenate(msgs, axis=0)
    partials = jnp.stack(parts, axis=0)
    updated = _node_mlp(node_features, partials, W1_n, W2_n)
    return (updated, messages)


# speedup vs baseline: 6.8838x; 1.1014x over previous
import jax
import jax.numpy as jnp
from jax import lax
from jax.experimental import pallas as pl
from jax.experimental.pallas import tpu as pltpu
from jax.experimental.pallas import tpu_sc as plsc

N_NODES = 10000
N_EDGES = 320000
D_FEAT = 128
D_EDGE = 16
HID = 128
N_FILTERS = 128

NC = 2
NS = 16
NW = NC * NS

NCHUNK = 5
EC = N_EDGES // NCHUNK


def _sc_mesh():
    return plsc.VectorSubcoreMesh(
        core_axis_name="c", subcore_axis_name="s", num_cores=NC, num_subcores=NS
    )


DPK = D_FEAT // 2
GB = EC // NS
GSUB = 80
GK = 5
GC = GSUB * GK
GNCH = GB // GC
EHALF = EC // 2


def _gather_body(koff, table_hbm, src_hbm, dst_hbm, a_hbm, b_hbm, idx_all,
                 rows, gsem0, gsem1, ssem0, ssem1):
    c = lax.axis_index("c")
    s = lax.axis_index("s")
    base = s * GB
    row0 = (s % (NS // 2)) * GB
    col0 = (s // (NS // 2)) * DPK
    gsems = (gsem0, gsem1)
    ssems = (ssem0, ssem1)

    def run(idx_hbm, out_hbm):
        pltpu.sync_copy(idx_hbm.at[pl.ds(koff + base, GB)], idx_all)

        def g_desc(i, b, k):
            off = i * GC + k * GSUB
            return pltpu.make_async_copy(
                table_hbm.at[idx_all.at[pl.ds(off, GSUB)]],
                rows.at[b, pl.ds(k * GSUB, GSUB)],
                gsems[b],
            )

        def s_desc(i, b):
            return pltpu.make_async_copy(
                rows.at[b],
                out_hbm.at[pl.ds(row0 + i * GC, GC), pl.ds(col0, DPK)],
                ssems[b],
            )

        for k in range(GK):
            g_desc(0, 0, k).start()

        def outer(j, carry):
            for b in range(2):
                i = 2 * j + b
                nb = 1 - b

                @pl.when(i >= 1)
                def _():
                    s_desc(i - 1, nb).wait()

                @pl.when(i + 1 < GNCH)
                def _():
                    for k in range(GK):
                        g_desc(i + 1, nb, k).start()

                for k in range(GK):
                    g_desc(i, b, k).wait()
                s_desc(i, b).start()
            return carry

        lax.fori_loop(0, GNCH // 2, outer, None)
        s_desc(GNCH - 1, (GNCH - 1) % 2).wait()

    @pl.when(c == 0)
    def _():
        run(src_hbm, a_hbm)

    @pl.when(c == 1)
    def _():
        run(dst_hbm, b_hbm)


def _sc_gather(nf_packed, src1d, dst1d, k):
    import functools
    f = pl.kernel(
        functools.partial(_gather_body, k * EC),
        out_type=(
            jax.ShapeDtypeStruct((EHALF, 2 * DPK), jnp.int32),
            jax.ShapeDtypeStruct((EHALF, 2 * DPK), jnp.int32),
        ),
        mesh=_sc_mesh(),
        compiler_params=pltpu.CompilerParams(use_tc_tiling_on_sc=False),
        scratch_types=[
            pltpu.VMEM((GB,), jnp.int32),
            pltpu.VMEM((2, GC, DPK), jnp.int32),
            pltpu.SemaphoreType.DMA,
            pltpu.SemaphoreType.DMA,
            pltpu.SemaphoreType.DMA,
            pltpu.SemaphoreType.DMA,
        ],
        name=f"edge_gather_{k}",
    )
    return f(nf_packed, src1d, dst1d)


SB = EC // NW
SSUB = 80
SKK = SB // SSUB
SUNR = 5
NPS = 624
NPS_LAST = N_NODES - (NS - 1) * NPS


def _scatter_body(koff, first, msg_hbm, src_hbm, dst_hbm, prev_hbm, out_hbm,
                  table, msgv, srcv, dstv, zbuf, scsem):
    c = lax.axis_index("c")
    s = lax.axis_index("s")
    w = c * NS + s
    ebase = w * SB

    pltpu.sync_copy(msg_hbm.at[pl.ds(ebase, SB)], msgv)
    pltpu.sync_copy(src_hbm.at[pl.ds(koff + ebase, SB)], srcv)
    pltpu.sync_copy(dst_hbm.at[pl.ds(koff + ebase, SB)], dstv)

    if first:
        def zloop(i, carry):
            zbuf[i, :] = jnp.zeros((D_EDGE,), jnp.float32)
            return carry

        lax.fori_loop(0, NPS_LAST, zloop, None)

        @pl.when(s < NS - 1)
        def _():
            pltpu.sync_copy(
                zbuf.at[pl.ds(0, NPS)], table.at[pl.ds(s * NPS, NPS)]
            )

        @pl.when(s == NS - 1)
        def _():
            pltpu.sync_copy(zbuf, table.at[pl.ds((NS - 1) * NPS, NPS_LAST)])
    else:
        @pl.when(s < NS - 1)
        def _():
            pltpu.sync_copy(
                prev_hbm.at[c, pl.ds(s * NPS, NPS)],
                table.at[pl.ds(s * NPS, NPS)],
            )

        @pl.when(s == NS - 1)
        def _():
            pltpu.sync_copy(
                prev_hbm.at[c, pl.ds((NS - 1) * NPS, NPS_LAST)],
                table.at[pl.ds((NS - 1) * NPS, NPS_LAST)],
            )

    plsc.subcore_barrier()

    def sub(jj, c2):
        for u in range(SUNR):
            j = jj * SUNR + u
            moff = j * SSUB
            pltpu.async_copy(
                msgv.at[pl.ds(moff, SSUB)],
                table.at[srcv.at[pl.ds(moff, SSUB)]], scsem, add=True,
            )
            pltpu.async_copy(
                msgv.at[pl.ds(moff, SSUB)],
                table.at[dstv.at[pl.ds(moff, SSUB)]], scsem, add=True,
            )
        return c2

    lax.fori_loop(0, SKK // SUNR, sub, None)

    def subw(jj, c2):
        for u in range(SUNR):
            j = jj * SUNR + u
            moff = j * SSUB
            pltpu.make_async_copy(
                msgv.at[pl.ds(moff, SSUB)],
                table.at[srcv.at[pl.ds(moff, SSUB)]], scsem,
            ).wait()
            pltpu.make_async_copy(
                msgv.at[pl.ds(moff, SSUB)],
                table.at[dstv.at[pl.ds(moff, SSUB)]], scsem,
            ).wait()
        return c2

    lax.fori_loop(0, SKK // SUNR, subw, None)
    plsc.subcore_barrier()

    @pl.when(s < NS - 1)
    def _():
        pltpu.sync_copy(
            table.at[pl.ds(s * NPS, NPS)], out_hbm.at[c, pl.ds(s * NPS, NPS)]
        )

    @pl.when(s == NS - 1)
    def _():
        pltpu.sync_copy(
            table.at[pl.ds((NS - 1) * NPS, NPS_LAST)],
            out_hbm.at[c, pl.ds((NS - 1) * NPS, NPS_LAST)],
        )


def _sc_scatter(messages_k, src1d, dst1d, prev, k):
    import functools
    f = pl.kernel(
        functools.partial(_scatter_body, k * EC, k == 0),
        out_type=jax.ShapeDtypeStruct((NC, N_NODES, D_EDGE), jnp.float32),
        mesh=_sc_mesh(),
        compiler_params=pltpu.CompilerParams(use_tc_tiling_on_sc=False),
        scratch_types=[
            pltpu.VMEM_SHARED((N_NODES, D_EDGE), jnp.float32),
            pltpu.VMEM((SB, D_EDGE), jnp.float32),
            pltpu.VMEM((SB,), jnp.int32),
            pltpu.VMEM((SB,), jnp.int32),
            pltpu.VMEM((NPS_LAST, D_EDGE), jnp.float32),
            pltpu.SemaphoreType.DMA,
        ],
        name=f"msg_scatter_{k}",
    )
    return f(messages_k, src1d, dst1d, prev)


EB = 1280
NBLK = EHALF // EB


def _unpack(x32):
    evens = lax.bitcast_convert_type(x32 << 16, jnp.float32)
    odds = lax.bitcast_convert_type(x32 & jnp.int32(-65536), jnp.float32)
    return jnp.concatenate([evens, odds], axis=1)


def _edge_mlp_half(a32, b32, ef, w1, w1b, w2):
    a = _unpack(a32)
    b = _unpack(b32)
    sym = (0.5 * (a + b)).astype(jnp.bfloat16)
    asym = (0.5 * jnp.abs(a - b)).astype(jnp.bfloat16)
    h = (
        jnp.dot(sym, w1b[0:D_FEAT], preferred_element_type=jnp.float32)
        + jnp.dot(asym, w1b[D_FEAT:2 * D_FEAT], preferred_element_type=jnp.float32)
        + jnp.dot(ef.astype(jnp.bfloat16),
                  w1b[2 * D_FEAT:2 * D_FEAT + D_EDGE],
                  preferred_element_type=jnp.float32)
        + w1[2 * D_FEAT + D_EDGE]
    )
    h = h * jax.nn.sigmoid(h)
    y = (
        jnp.dot(h.astype(jnp.bfloat16), w2.astype(jnp.bfloat16)[:HID],
                preferred_element_type=jnp.float32)
        + w2[HID]
    )
    return y * jax.nn.sigmoid(y)


def _edge_mlp_body(a_ref, b_ref, e_ref, w1_ref, w2_ref, out_ref):
    a32 = a_ref[...]
    b32 = b_ref[...]
    ef = e_ref[...]
    w1 = w1_ref[...]
    w1b = w1.astype(jnp.bfloat16)
    w2 = w2_ref[...]
    out_ref[0] = _edge_mlp_half(
        a32[:, :DPK], b32[:, :DPK], ef[0], w1, w1b, w2
    )
    out_ref[1] = _edge_mlp_half(
        a32[:, DPK:], b32[:, DPK:], ef[1], w1, w1b, w2
    )


def _edge_mlp(a_rows, b_rows, ef_k2, W1p, W2_e):
    out = pl.pallas_call(
        _edge_mlp_body,
        grid=(NBLK,),
        in_specs=[
            pl.BlockSpec((EB, 2 * DPK), lambda i: (i, 0)),
            pl.BlockSpec((EB, 2 * DPK), lambda i: (i, 0)),
            pl.BlockSpec((2, EB, D_EDGE), lambda i: (0, i, 0)),
            pl.BlockSpec(W1p.shape, lambda i: (0, 0)),
            pl.BlockSpec(W2_e.shape, lambda i: (0, 0)),
        ],
        out_specs=pl.BlockSpec((2, EB, D_EDGE), lambda i: (0, i, 0)),
        out_shape=jax.ShapeDtypeStruct((2, EHALF, D_EDGE), jnp.float32),
    )(a_rows, b_rows, ef_k2, W1p, W2_e)
    return out.reshape(EC, D_EDGE)


NB = 2000


def _node_mlp_body(nf_ref, p_ref, w1_ref, w2_ref, out_ref):
    upd = p_ref[0] + p_ref[1]
    x = nf_ref[...]
    w1 = w1_ref[...]
    h = (
        jnp.dot(x, w1[0:D_FEAT], preferred_element_type=jnp.float32)
        + jnp.dot(upd, w1[D_FEAT:D_FEAT + D_EDGE],
                  preferred_element_type=jnp.float32)
        + w1[D_FEAT + D_EDGE]
    )
    h = h * jax.nn.sigmoid(h)
    w2 = w2_ref[...]
    y = jnp.dot(h, w2[:HID], preferred_element_type=jnp.float32) + w2[HID]
    out_ref[...] = y * jax.nn.sigmoid(y)


def _node_mlp(node_features, partials, W1_n, W2_n):
    return pl.pallas_call(
        _node_mlp_body,
        grid=(N_NODES // NB,),
        in_specs=[
            pl.BlockSpec((NB, D_FEAT), lambda i: (i, 0)),
            pl.BlockSpec((NC, NB, D_EDGE), lambda i: (0, i, 0)),
            pl.BlockSpec(W1_n.shape, lambda i: (0, 0)),
            pl.BlockSpec(W2_n.shape, lambda i: (0, 0)),
        ],
        out_specs=pl.BlockSpec((NB, N_FILTERS), lambda i: (i, 0)),
        out_shape=jax.ShapeDtypeStruct((N_NODES, N_FILTERS), jnp.float32),
    )(node_features, partials, W1_n, W2_n)


def kernel(node_features, edge_features, edges, W1_e, W2_e, W1_n, W2_n):
    src1d = edges[:, 0]
    dst1d = edges[:, 1]
    nf_packed = lax.bitcast_convert_type(
        node_features.astype(jnp.bfloat16).reshape(N_NODES, DPK, 2), jnp.int32
    )
    perm = jnp.concatenate(
        [jnp.arange(0, D_FEAT, 2), jnp.arange(1, D_FEAT, 2)]
    )
    W1p = jnp.concatenate(
        [W1_e[0:D_FEAT][perm], W1_e[D_FEAT:2 * D_FEAT][perm],
         W1_e[2 * D_FEAT:]], axis=0
    )
    msgs = []
    partials = jnp.zeros((NC, N_NODES, D_EDGE), jnp.float32)
    for k in range(NCHUNK):
        a_rows, b_rows = _sc_gather(nf_packed, src1d, dst1d, k)
        ef_k = lax.slice(edge_features, (k * EC, 0), ((k + 1) * EC, D_EDGE))
        msg_k = _edge_mlp(a_rows, b_rows, ef_k.reshape(2, EHALF, D_EDGE),
                          W1p, W2_e)
        msgs.append(msg_k)
        partials = _sc_scatter(msg_k, src1d, dst1d, partials, k)
    messages = jnp.concatenate(msgs, axis=0)
    updated = _node_mlp(node_features, partials, W1_n, W2_n)
    return (updated, messages)


# speedup vs baseline: 7.1277x; 1.0354x over previous
import jax
import jax.numpy as jnp
from jax import lax
from jax.experimental import pallas as pl
from jax.experimental.pallas import tpu as pltpu
from jax.experimental.pallas import tpu_sc as plsc

N_NODES = 10000
N_EDGES = 320000
D_FEAT = 128
D_EDGE = 16
HID = 128
N_FILTERS = 128

NC = 2
NS = 16
NW = NC * NS

NCHUNK = 5
EC = N_EDGES // NCHUNK


def _sc_mesh():
    return plsc.VectorSubcoreMesh(
        core_axis_name="c", subcore_axis_name="s", num_cores=NC, num_subcores=NS
    )


DPK = D_FEAT // 2
GB = EC // NS
GSUB = 80
GK = 5
GC = GSUB * GK
GNCH = GB // GC
EHALF = EC // 2


def _gather_body(koff, table_hbm, src_hbm, dst_hbm, a_hbm, b_hbm, idx_all,
                 rows, gsem0, gsem1, ssem0, ssem1):
    c = lax.axis_index("c")
    s = lax.axis_index("s")
    base = s * GB
    row0 = (s % (NS // 2)) * GB
    col0 = (s // (NS // 2)) * DPK
    gsems = (gsem0, gsem1)
    ssems = (ssem0, ssem1)

    def run(idx_hbm, out_hbm):
        pltpu.sync_copy(idx_hbm.at[pl.ds(koff + base, GB)], idx_all)

        def g_desc(i, b, k):
            off = i * GC + k * GSUB
            return pltpu.make_async_copy(
                table_hbm.at[idx_all.at[pl.ds(off, GSUB)]],
                rows.at[b, pl.ds(k * GSUB, GSUB)],
                gsems[b],
            )

        def s_desc(i, b):
            return pltpu.make_async_copy(
                rows.at[b],
                out_hbm.at[pl.ds(row0 + i * GC, GC), pl.ds(col0, DPK)],
                ssems[b],
            )

        for k in range(GK):
            g_desc(0, 0, k).start()

        def outer(j, carry):
            for b in range(2):
                i = 2 * j + b
                nb = 1 - b

                @pl.when(i >= 1)
                def _():
                    s_desc(i - 1, nb).wait()

                @pl.when(i + 1 < GNCH)
                def _():
                    for k in range(GK):
                        g_desc(i + 1, nb, k).start()

                for k in range(GK):
                    g_desc(i, b, k).wait()
                s_desc(i, b).start()
            return carry

        lax.fori_loop(0, GNCH // 2, outer, None)
        s_desc(GNCH - 1, (GNCH - 1) % 2).wait()

    @pl.when(c == 0)
    def _():
        run(src_hbm, a_hbm)

    @pl.when(c == 1)
    def _():
        run(dst_hbm, b_hbm)


def _sc_gather(nf_packed, src1d, dst1d, k):
    import functools
    f = pl.kernel(
        functools.partial(_gather_body, k * EC),
        out_type=(
            jax.ShapeDtypeStruct((EHALF, 2 * DPK), jnp.int32),
            jax.ShapeDtypeStruct((EHALF, 2 * DPK), jnp.int32),
        ),
        mesh=_sc_mesh(),
        compiler_params=pltpu.CompilerParams(use_tc_tiling_on_sc=False),
        scratch_types=[
            pltpu.VMEM((GB,), jnp.int32),
            pltpu.VMEM((2, GC, DPK), jnp.int32),
            pltpu.SemaphoreType.DMA,
            pltpu.SemaphoreType.DMA,
            pltpu.SemaphoreType.DMA,
            pltpu.SemaphoreType.DMA,
        ],
        name=f"edge_gather_{k}",
    )
    return f(nf_packed, src1d, dst1d)


SB = EC // NW
SSUB = 80
SKK = SB // SSUB
SUNR = 5
NPS = 624
NPS_LAST = N_NODES - (NS - 1) * NPS


def _scatter_body(koff, first, msg_hbm, src_hbm, dst_hbm, prev_hbm, out_hbm,
                  table, msgv, srcv, dstv, zbuf, scsem):
    c = lax.axis_index("c")
    s = lax.axis_index("s")
    w = c * NS + s
    ebase = w * SB

    pltpu.sync_copy(msg_hbm.at[pl.ds(ebase, SB)], msgv)
    pltpu.sync_copy(src_hbm.at[pl.ds(koff + ebase, SB)], srcv)
    pltpu.sync_copy(dst_hbm.at[pl.ds(koff + ebase, SB)], dstv)

    if first:
        def zloop(i, carry):
            zbuf[i, :] = jnp.zeros((D_EDGE,), jnp.float32)
            return carry

        lax.fori_loop(0, NPS_LAST, zloop, None)

        @pl.when(s < NS - 1)
        def _():
            pltpu.sync_copy(
                zbuf.at[pl.ds(0, NPS)], table.at[pl.ds(s * NPS, NPS)]
            )

        @pl.when(s == NS - 1)
        def _():
            pltpu.sync_copy(zbuf, table.at[pl.ds((NS - 1) * NPS, NPS_LAST)])
    else:
        @pl.when(s < NS - 1)
        def _():
            pltpu.sync_copy(
                prev_hbm.at[c, pl.ds(s * NPS, NPS)],
                table.at[pl.ds(s * NPS, NPS)],
            )

        @pl.when(s == NS - 1)
        def _():
            pltpu.sync_copy(
                prev_hbm.at[c, pl.ds((NS - 1) * NPS, NPS_LAST)],
                table.at[pl.ds((NS - 1) * NPS, NPS_LAST)],
            )

    plsc.subcore_barrier()

    def sub(jj, c2):
        for u in range(SUNR):
            j = jj * SUNR + u
            moff = j * SSUB
            pltpu.async_copy(
                msgv.at[pl.ds(moff, SSUB)],
                table.at[srcv.at[pl.ds(moff, SSUB)]], scsem, add=True,
            )
            pltpu.async_copy(
                msgv.at[pl.ds(moff, SSUB)],
                table.at[dstv.at[pl.ds(moff, SSUB)]], scsem, add=True,
            )
        return c2

    lax.fori_loop(0, SKK // SUNR, sub, None)

    def subw(jj, c2):
        for u in range(SUNR):
            j = jj * SUNR + u
            moff = j * SSUB
            pltpu.make_async_copy(
                msgv.at[pl.ds(moff, SSUB)],
                table.at[srcv.at[pl.ds(moff, SSUB)]], scsem,
            ).wait()
            pltpu.make_async_copy(
                msgv.at[pl.ds(moff, SSUB)],
                table.at[dstv.at[pl.ds(moff, SSUB)]], scsem,
            ).wait()
        return c2

    lax.fori_loop(0, SKK // SUNR, subw, None)
    plsc.subcore_barrier()

    @pl.when(s < NS - 1)
    def _():
        pltpu.sync_copy(
            table.at[pl.ds(s * NPS, NPS)], out_hbm.at[c, pl.ds(s * NPS, NPS)]
        )

    @pl.when(s == NS - 1)
    def _():
        pltpu.sync_copy(
            table.at[pl.ds((NS - 1) * NPS, NPS_LAST)],
            out_hbm.at[c, pl.ds((NS - 1) * NPS, NPS_LAST)],
        )


def _sc_scatter(messages_k, src1d, dst1d, prev, k):
    import functools
    f = pl.kernel(
        functools.partial(_scatter_body, k * EC, k == 0),
        out_type=jax.ShapeDtypeStruct((NC, N_NODES, D_EDGE), jnp.float32),
        mesh=_sc_mesh(),
        compiler_params=pltpu.CompilerParams(use_tc_tiling_on_sc=False),
        scratch_types=[
            pltpu.VMEM_SHARED((N_NODES, D_EDGE), jnp.float32),
            pltpu.VMEM((SB, D_EDGE), jnp.float32),
            pltpu.VMEM((SB,), jnp.int32),
            pltpu.VMEM((SB,), jnp.int32),
            pltpu.VMEM((NPS_LAST, D_EDGE), jnp.float32),
            pltpu.SemaphoreType.DMA,
        ],
        name=f"msg_scatter_{k}",
    )
    return f(messages_k, src1d, dst1d, prev)


EB = 1280
NBLK = EHALF // EB


def _unpack(x32):
    evens = lax.bitcast_convert_type(x32 << 16, jnp.float32)
    odds = lax.bitcast_convert_type(x32 & jnp.int32(-65536), jnp.float32)
    return jnp.concatenate([evens, odds], axis=1)


def _edge_mlp_half(a32, b32, ef, w1, w1b, w2):
    a = _unpack(a32)
    b = _unpack(b32)
    sym = (0.5 * (a + b)).astype(jnp.bfloat16)
    asym = (0.5 * jnp.abs(a - b)).astype(jnp.bfloat16)
    h = (
        jnp.dot(sym, w1b[0:D_FEAT], preferred_element_type=jnp.float32)
        + jnp.dot(asym, w1b[D_FEAT:2 * D_FEAT], preferred_element_type=jnp.float32)
        + jnp.dot(ef.astype(jnp.bfloat16),
                  w1b[2 * D_FEAT:2 * D_FEAT + D_EDGE],
                  preferred_element_type=jnp.float32)
        + w1[2 * D_FEAT + D_EDGE]
    )
    h = h * jax.nn.sigmoid(h)
    y = (
        jnp.dot(h.astype(jnp.bfloat16), w2.astype(jnp.bfloat16)[:HID],
                preferred_element_type=jnp.float32)
        + w2[HID]
    )
    return y * jax.nn.sigmoid(y)


def _edge_mlp_body(a_ref, b_ref, e_ref, w1_ref, w2_ref, big_ref, out_ref):
    a32 = a_ref[...]
    b32 = b_ref[...]
    ef = e_ref[...]
    w1 = w1_ref[...]
    w1b = w1.astype(jnp.bfloat16)
    w2 = w2_ref[...]
    m0 = _edge_mlp_half(a32[:, :DPK], b32[:, :DPK], ef[0], w1, w1b, w2)
    m1 = _edge_mlp_half(a32[:, DPK:], b32[:, DPK:], ef[1], w1, w1b, w2)
    big_ref[0] = m0
    big_ref[1] = m1
    out_ref[0] = m0
    out_ref[1] = m1


def _edge_mlp(a_rows, b_rows, ef_k2, W1p, W2_e, msgbuf, k):
    in_specs = [
        pl.BlockSpec((EB, 2 * DPK), lambda i: (i, 0)),
        pl.BlockSpec((EB, 2 * DPK), lambda i: (i, 0)),
        pl.BlockSpec((2, EB, D_EDGE), lambda i: (0, i, 0)),
        pl.BlockSpec(W1p.shape, lambda i: (0, 0)),
        pl.BlockSpec(W2_e.shape, lambda i: (0, 0)),
    ]
    args = [a_rows, b_rows, ef_k2, W1p, W2_e]
    aliases = {}
    if msgbuf is not None:
        in_specs.append(pl.BlockSpec(memory_space=pltpu.MemorySpace.HBM))
        args.append(msgbuf)
        aliases = {5: 0}
    big, out = pl.pallas_call(
        _edge_mlp_body_alias if msgbuf is not None else _edge_mlp_body,
        grid=(NBLK,),
        in_specs=in_specs,
        out_specs=[
            pl.BlockSpec((2, EB, D_EDGE), lambda i, k=k: (k, i, 0)),
            pl.BlockSpec((2, EB, D_EDGE), lambda i: (0, i, 0)),
        ],
        out_shape=[
            jax.ShapeDtypeStruct((2 * NCHUNK, EHALF, D_EDGE), jnp.float32),
            jax.ShapeDtypeStruct((2, EHALF, D_EDGE), jnp.float32),
        ],
        input_output_aliases=aliases,
    )(*args)
    return big, out.reshape(EC, D_EDGE)


def _edge_mlp_body_alias(a_ref, b_ref, e_ref, w1_ref, w2_ref, mbuf_ref,
                         big_ref, out_ref):
    del mbuf_ref
    _edge_mlp_body(a_ref, b_ref, e_ref, w1_ref, w2_ref, big_ref, out_ref)


NB = 2000


def _node_mlp_body(nf_ref, p_ref, w1_ref, w2_ref, out_ref):
    upd = p_ref[0] + p_ref[1]
    x = nf_ref[...]
    w1 = w1_ref[...]
    h = (
        jnp.dot(x, w1[0:D_FEAT], preferred_element_type=jnp.float32)
        + jnp.dot(upd, w1[D_FEAT:D_FEAT + D_EDGE],
                  preferred_element_type=jnp.float32)
        + w1[D_FEAT + D_EDGE]
    )
    h = h * jax.nn.sigmoid(h)
    w2 = w2_ref[...]
    y = jnp.dot(h, w2[:HID], preferred_element_type=jnp.float32) + w2[HID]
    out_ref[...] = y * jax.nn.sigmoid(y)


def _node_mlp(node_features, partials, W1_n, W2_n):
    return pl.pallas_call(
        _node_mlp_body,
        grid=(N_NODES // NB,),
        in_specs=[
            pl.BlockSpec((NB, D_FEAT), lambda i: (i, 0)),
            pl.BlockSpec((NC, NB, D_EDGE), lambda i: (0, i, 0)),
            pl.BlockSpec(W1_n.shape, lambda i: (0, 0)),
            pl.BlockSpec(W2_n.shape, lambda i: (0, 0)),
        ],
        out_specs=pl.BlockSpec((NB, N_FILTERS), lambda i: (i, 0)),
        out_shape=jax.ShapeDtypeStruct((N_NODES, N_FILTERS), jnp.float32),
    )(node_features, partials, W1_n, W2_n)


def kernel(node_features, edge_features, edges, W1_e, W2_e, W1_n, W2_n):
    src1d = edges[:, 0]
    dst1d = edges[:, 1]
    nf_packed = lax.bitcast_convert_type(
        node_features.astype(jnp.bfloat16).reshape(N_NODES, DPK, 2), jnp.int32
    )
    perm = jnp.concatenate(
        [jnp.arange(0, D_FEAT, 2), jnp.arange(1, D_FEAT, 2)]
    )
    W1p = jnp.concatenate(
        [W1_e[0:D_FEAT][perm], W1_e[D_FEAT:2 * D_FEAT][perm],
         W1_e[2 * D_FEAT:]], axis=0
    )
    msgbuf = None
    partials = jnp.zeros((NC, N_NODES, D_EDGE), jnp.float32)
    for k in range(NCHUNK):
        a_rows, b_rows = _sc_gather(nf_packed, src1d, dst1d, k)
        ef_k = lax.slice(edge_features, (k * EC, 0), ((k + 1) * EC, D_EDGE))
        msgbuf, msg_k = _edge_mlp(a_rows, b_rows,
                                  ef_k.reshape(2, EHALF, D_EDGE),
                                  W1p, W2_e, msgbuf, k)
        partials = _sc_scatter(msg_k, src1d, dst1d, partials, k)
    messages = msgbuf.reshape(N_EDGES, D_EDGE)
    updated = _node_mlp(node_features, partials, W1_n, W2_n)
    return (updated, messages)


# speedup vs baseline: 7.4136x; 1.0401x over previous
import jax
import jax.numpy as jnp
from jax import lax
from jax.experimental import pallas as pl
from jax.experimental.pallas import tpu as pltpu
from jax.experimental.pallas import tpu_sc as plsc

N_NODES = 10000
N_EDGES = 320000
D_FEAT = 128
D_EDGE = 16
HID = 128
N_FILTERS = 128

NC = 2
NS = 16
NW = NC * NS

NCHUNK = 5
EC = N_EDGES // NCHUNK


def _sc_mesh():
    return plsc.VectorSubcoreMesh(
        core_axis_name="c", subcore_axis_name="s", num_cores=NC, num_subcores=NS
    )


DPK = D_FEAT // 2
GB = EC // NS
GSUB = 80
GK = 5
GC = GSUB * GK
GNCH = GB // GC
EHALF = EC // 2


def _gather_body(koff, table_hbm, src_hbm, dst_hbm, a_hbm, b_hbm, idx_all,
                 rows, gsem0, gsem1, ssem0, ssem1):
    c = lax.axis_index("c")
    s = lax.axis_index("s")
    base = s * GB
    row0 = (s % (NS // 2)) * GB
    col0 = (s // (NS // 2)) * DPK
    gsems = (gsem0, gsem1)
    ssems = (ssem0, ssem1)

    def run(idx_hbm, out_hbm):
        pltpu.sync_copy(idx_hbm.at[pl.ds(koff + base, GB)], idx_all)

        def g_desc(i, b, k):
            off = i * GC + k * GSUB
            return pltpu.make_async_copy(
                table_hbm.at[idx_all.at[pl.ds(off, GSUB)]],
                rows.at[b, pl.ds(k * GSUB, GSUB)],
                gsems[b],
            )

        def s_desc(i, b):
            return pltpu.make_async_copy(
                rows.at[b],
                out_hbm.at[pl.ds(row0 + i * GC, GC), pl.ds(col0, DPK)],
                ssems[b],
            )

        for k in range(GK):
            g_desc(0, 0, k).start()

        def outer(j, carry):
            for b in range(2):
                i = 2 * j + b
                nb = 1 - b

                @pl.when(i >= 1)
                def _():
                    s_desc(i - 1, nb).wait()

                @pl.when(i + 1 < GNCH)
                def _():
                    for k in range(GK):
                        g_desc(i + 1, nb, k).start()

                for k in range(GK):
                    g_desc(i, b, k).wait()
                s_desc(i, b).start()
            return carry

        lax.fori_loop(0, GNCH // 2, outer, None)
        s_desc(GNCH - 1, (GNCH - 1) % 2).wait()

    @pl.when(c == 0)
    def _():
        run(src_hbm, a_hbm)

    @pl.when(c == 1)
    def _():
        run(dst_hbm, b_hbm)


def _sc_gather(nf_packed, src1d, dst1d, k):
    import functools
    f = pl.kernel(
        functools.partial(_gather_body, k * EC),
        out_type=(
            jax.ShapeDtypeStruct((EHALF, 2 * DPK), jnp.int32),
            jax.ShapeDtypeStruct((EHALF, 2 * DPK), jnp.int32),
        ),
        mesh=_sc_mesh(),
        compiler_params=pltpu.CompilerParams(use_tc_tiling_on_sc=False),
        scratch_types=[
            pltpu.VMEM((GB,), jnp.int32),
            pltpu.VMEM((2, GC, DPK), jnp.int32),
            pltpu.SemaphoreType.DMA,
            pltpu.SemaphoreType.DMA,
            pltpu.SemaphoreType.DMA,
            pltpu.SemaphoreType.DMA,
        ],
        name=f"edge_gather_{k}",
    )
    return f(nf_packed, src1d, dst1d)


SB = EC // NW
SSUB = 80
SKK = SB // SSUB
SUNR = 5
NPS = 624
NPS_LAST = N_NODES - (NS - 1) * NPS


def _scatter_body(koff, first, msg_hbm, src_hbm, dst_hbm, prev_hbm, out_hbm,
                  table, msgv, srcv, dstv, zbuf, scsem):
    c = lax.axis_index("c")
    s = lax.axis_index("s")
    w = c * NS + s
    ebase = w * SB

    pltpu.sync_copy(msg_hbm.at[pl.ds(ebase, SB)], msgv)
    pltpu.sync_copy(src_hbm.at[pl.ds(koff + ebase, SB)], srcv)
    pltpu.sync_copy(dst_hbm.at[pl.ds(koff + ebase, SB)], dstv)

    if first:
        def zloop(i, carry):
            zbuf[i, :] = jnp.zeros((D_EDGE,), jnp.float32)
            return carry

        lax.fori_loop(0, NPS_LAST, zloop, None)

        @pl.when(s < NS - 1)
        def _():
            pltpu.sync_copy(
                zbuf.at[pl.ds(0, NPS)], table.at[pl.ds(s * NPS, NPS)]
            )

        @pl.when(s == NS - 1)
        def _():
            pltpu.sync_copy(zbuf, table.at[pl.ds((NS - 1) * NPS, NPS_LAST)])
    else:
        @pl.when(s < NS - 1)
        def _():
            pltpu.sync_copy(
                prev_hbm.at[c, pl.ds(s * NPS, NPS)],
                table.at[pl.ds(s * NPS, NPS)],
            )

        @pl.when(s == NS - 1)
        def _():
            pltpu.sync_copy(
                prev_hbm.at[c, pl.ds((NS - 1) * NPS, NPS_LAST)],
                table.at[pl.ds((NS - 1) * NPS, NPS_LAST)],
            )

    plsc.subcore_barrier()

    def sub(jj, c2):
        for u in range(SUNR):
            j = jj * SUNR + u
            moff = j * SSUB
            pltpu.async_copy(
                msgv.at[pl.ds(moff, SSUB)],
                table.at[srcv.at[pl.ds(moff, SSUB)]], scsem, add=True,
            )
            pltpu.async_copy(
                msgv.at[pl.ds(moff, SSUB)],
                table.at[dstv.at[pl.ds(moff, SSUB)]], scsem, add=True,
            )
        return c2

    lax.fori_loop(0, SKK // SUNR, sub, None)

    def subw(jj, c2):
        for u in range(SUNR):
            j = jj * SUNR + u
            moff = j * SSUB
            pltpu.make_async_copy(
                msgv.at[pl.ds(moff, SSUB)],
                table.at[srcv.at[pl.ds(moff, SSUB)]], scsem,
            ).wait()
            pltpu.make_async_copy(
                msgv.at[pl.ds(moff, SSUB)],
                table.at[dstv.at[pl.ds(moff, SSUB)]], scsem,
            ).wait()
        return c2

    lax.fori_loop(0, SKK // SUNR, subw, None)
    plsc.subcore_barrier()

    @pl.when(s < NS - 1)
    def _():
        pltpu.sync_copy(
            table.at[pl.ds(s * NPS, NPS)], out_hbm.at[c, pl.ds(s * NPS, NPS)]
        )

    @pl.when(s == NS - 1)
    def _():
        pltpu.sync_copy(
            table.at[pl.ds((NS - 1) * NPS, NPS_LAST)],
            out_hbm.at[c, pl.ds((NS - 1) * NPS, NPS_LAST)],
        )


def _sc_scatter(messages_k, src1d, dst1d, prev, k):
    import functools
    f = pl.kernel(
        functools.partial(_scatter_body, k * EC, k == 0),
        out_type=jax.ShapeDtypeStruct((NC, N_NODES, D_EDGE), jnp.float32),
        mesh=_sc_mesh(),
        compiler_params=pltpu.CompilerParams(use_tc_tiling_on_sc=False),
        scratch_types=[
            pltpu.VMEM_SHARED((N_NODES, D_EDGE), jnp.float32),
            pltpu.VMEM((SB, D_EDGE), jnp.float32),
            pltpu.VMEM((SB,), jnp.int32),
            pltpu.VMEM((SB,), jnp.int32),
            pltpu.VMEM((NPS_LAST, D_EDGE), jnp.float32),
            pltpu.SemaphoreType.DMA,
        ],
        name=f"msg_scatter_{k}",
    )
    return f(messages_k, src1d, dst1d, prev)


EB = 3200
NBLK = EHALF // EB


def _unpack(x32):
    evens = lax.bitcast_convert_type(x32 << 16, jnp.float32)
    odds = lax.bitcast_convert_type(x32 & jnp.int32(-65536), jnp.float32)
    return jnp.concatenate([evens, odds], axis=1)


def _edge_mlp_half(a32, b32, ef, w1, w1b, w2):
    a = _unpack(a32)
    b = _unpack(b32)
    sym = (0.5 * (a + b)).astype(jnp.bfloat16)
    asym = (0.5 * jnp.abs(a - b)).astype(jnp.bfloat16)
    h = (
        jnp.dot(sym, w1b[0:D_FEAT], preferred_element_type=jnp.float32)
        + jnp.dot(asym, w1b[D_FEAT:2 * D_FEAT], preferred_element_type=jnp.float32)
        + jnp.dot(ef.astype(jnp.bfloat16),
                  w1b[2 * D_FEAT:2 * D_FEAT + D_EDGE],
                  preferred_element_type=jnp.float32)
        + w1[2 * D_FEAT + D_EDGE]
    )
    h = h * jax.nn.sigmoid(h)
    y = (
        jnp.dot(h.astype(jnp.bfloat16), w2.astype(jnp.bfloat16)[:HID],
                preferred_element_type=jnp.float32)
        + w2[HID]
    )
    return y * jax.nn.sigmoid(y)


def _edge_mlp_body(a_ref, b_ref, e_ref, w1_ref, w2_ref, big_ref, out_ref):
    a32 = a_ref[...]
    b32 = b_ref[...]
    ef = e_ref[...]
    w1 = w1_ref[...]
    w1b = w1.astype(jnp.bfloat16)
    w2 = w2_ref[...]
    m0 = _edge_mlp_half(a32[:, :DPK], b32[:, :DPK], ef[0], w1, w1b, w2)
    m1 = _edge_mlp_half(a32[:, DPK:], b32[:, DPK:], ef[1], w1, w1b, w2)
    big_ref[0] = m0
    big_ref[1] = m1
    out_ref[0] = m0
    out_ref[1] = m1


def _edge_mlp(a_rows, b_rows, ef_k2, W1p, W2_e, msgbuf, k):
    in_specs = [
        pl.BlockSpec((EB, 2 * DPK), lambda i: (i, 0)),
        pl.BlockSpec((EB, 2 * DPK), lambda i: (i, 0)),
        pl.BlockSpec((2, EB, D_EDGE), lambda i: (0, i, 0)),
        pl.BlockSpec(W1p.shape, lambda i: (0, 0)),
        pl.BlockSpec(W2_e.shape, lambda i: (0, 0)),
    ]
    args = [a_rows, b_rows, ef_k2, W1p, W2_e]
    aliases = {}
    if msgbuf is not None:
        in_specs.append(pl.BlockSpec(memory_space=pltpu.MemorySpace.HBM))
        args.append(msgbuf)
        aliases = {5: 0}
    big, out = pl.pallas_call(
        _edge_mlp_body_alias if msgbuf is not None else _edge_mlp_body,
        grid=(NBLK,),
        in_specs=in_specs,
        out_specs=[
            pl.BlockSpec((2, EB, D_EDGE), lambda i, k=k: (k, i, 0)),
            pl.BlockSpec((2, EB, D_EDGE), lambda i: (0, i, 0)),
        ],
        out_shape=[
            jax.ShapeDtypeStruct((2 * NCHUNK, EHALF, D_EDGE), jnp.float32),
            jax.ShapeDtypeStruct((2, EHALF, D_EDGE), jnp.float32),
        ],
        input_output_aliases=aliases,
    )(*args)
    return big, out.reshape(EC, D_EDGE)


def _edge_mlp_body_alias(a_ref, b_ref, e_ref, w1_ref, w2_ref, mbuf_ref,
                         big_ref, out_ref):
    del mbuf_ref
    _edge_mlp_body(a_ref, b_ref, e_ref, w1_ref, w2_ref, big_ref, out_ref)


NB = 2000


def _node_mlp_body(nf_ref, p_ref, w1_ref, w2_ref, out_ref):
    upd = p_ref[0] + p_ref[1]
    x = nf_ref[...]
    w1 = w1_ref[...]
    h = (
        jnp.dot(x, w1[0:D_FEAT], preferred_element_type=jnp.float32)
        + jnp.dot(upd, w1[D_FEAT:D_FEAT + D_EDGE],
                  preferred_element_type=jnp.float32)
        + w1[D_FEAT + D_EDGE]
    )
    h = h * jax.nn.sigmoid(h)
    w2 = w2_ref[...]
    y = jnp.dot(h, w2[:HID], preferred_element_type=jnp.float32) + w2[HID]
    out_ref[...] = y * jax.nn.sigmoid(y)


def _node_mlp(node_features, partials, W1_n, W2_n):
    return pl.pallas_call(
        _node_mlp_body,
        grid=(N_NODES // NB,),
        in_specs=[
            pl.BlockSpec((NB, D_FEAT), lambda i: (i, 0)),
            pl.BlockSpec((NC, NB, D_EDGE), lambda i: (0, i, 0)),
            pl.BlockSpec(W1_n.shape, lambda i: (0, 0)),
            pl.BlockSpec(W2_n.shape, lambda i: (0, 0)),
        ],
        out_specs=pl.BlockSpec((NB, N_FILTERS), lambda i: (i, 0)),
        out_shape=jax.ShapeDtypeStruct((N_NODES, N_FILTERS), jnp.float32),
    )(node_features, partials, W1_n, W2_n)


def kernel(node_features, edge_features, edges, W1_e, W2_e, W1_n, W2_n):
    src1d = edges[:, 0]
    dst1d = edges[:, 1]
    nf_packed = lax.bitcast_convert_type(
        node_features.astype(jnp.bfloat16).reshape(N_NODES, DPK, 2), jnp.int32
    )
    perm = jnp.concatenate(
        [jnp.arange(0, D_FEAT, 2), jnp.arange(1, D_FEAT, 2)]
    )
    W1p = jnp.concatenate(
        [W1_e[0:D_FEAT][perm], W1_e[D_FEAT:2 * D_FEAT][perm],
         W1_e[2 * D_FEAT:]], axis=0
    )
    msgbuf = None
    partials = jnp.zeros((NC, N_NODES, D_EDGE), jnp.float32)
    for k in range(NCHUNK):
        a_rows, b_rows = _sc_gather(nf_packed, src1d, dst1d, k)
        ef_k = lax.slice(edge_features, (k * EC, 0), ((k + 1) * EC, D_EDGE))
        msgbuf, msg_k = _edge_mlp(a_rows, b_rows,
                                  ef_k.reshape(2, EHALF, D_EDGE),
                                  W1p, W2_e, msgbuf, k)
        partials = _sc_scatter(msg_k, src1d, dst1d, partials, k)
    messages = msgbuf.reshape(N_EDGES, D_EDGE)
    updated = _node_mlp(node_features, partials, W1_n, W2_n)
    return (updated, messages)
